# baseline (speedup 1.0000x reference)
import jax
import jax.numpy as jnp
from jax.experimental import pallas as pl
from jax.experimental.pallas import tpu as pltpu
from jax.experimental.pallas import tpu_sc as plsc

_LANES = 128
_WIDTH = 2048
_BLOCK_ROWS = 256
_SC_LANES = 16


def _sc_forward(x2, packed, params_f, params_i):
    rows = x2.shape[0]
    mesh = plsc.VectorSubcoreMesh(core_axis_name="c", subcore_axis_name="s")

    @pl.kernel(
        out_type=jax.ShapeDtypeStruct((rows, _WIDTH), jnp.float32),
        mesh=mesh,
        scratch_types=[
            pltpu.VMEM((256,), jnp.int32),
            pltpu.VMEM((2, _SC_LANES), jnp.float32),
            pltpu.VMEM((2, _SC_LANES), jnp.int32),
            pltpu.SemaphoreType.DMA,
        ],
        compiler_params=pltpu.CompilerParams(needs_layout_passes=False),
    )
    def sck(tab_hbm, pf_hbm, pi_hbm, x_hbm, o_hbm, tab_v, pf_v, pi_v, sem):
        pltpu.async_copy(tab_hbm, tab_v, sem).wait()
        pltpu.async_copy(pf_hbm, pf_v, sem).wait()
        pltpu.async_copy(pi_hbm, pi_v, sem).wait()
        inv_s = pf_v[0, :]
        two_qq = pf_v[1, :]
        sh1 = pi_v[0, :]
        sh2 = pi_v[1, :]

        def body(x_v, o_v):
            @pl.loop(0, 8)
            def _(rr):
                @pl.loop(0, _LANES, step=_SC_LANES)
                def _(cc):
                    xv = x_v[rr, pl.ds(cc, _SC_LANES)]
                    t = xv * inv_s
                    t = jnp.minimum(jnp.maximum(t, -32768.0), 32767.0)
                    qi = t.astype(jnp.int32)
                    fb = qi.astype(jnp.float32)
                    qi = jnp.where(fb > t, qi - 1, qi)
                    idx = (qi >> 8) + 128
                    r = qi & 255
                    p = plsc.load_gather(tab_v, [idx])
                    m1 = (p << 16) >> 16
                    v = (r * m1) >> sh1
                    w = ((p >> 16) << sh2) + v
                    o_v[rr, pl.ds(cc, _SC_LANES)] = w.astype(jnp.float32) * two_qq

        pltpu.emit_pipeline(
            body,
            grid=(rows // 8, _WIDTH // _LANES),
            in_specs=[pl.BlockSpec((8, _LANES), index_map=lambda i, j: (i, j))],
            out_specs=[pl.BlockSpec((8, _LANES), index_map=lambda i, j: (i, j))],
            core_axis_name=("c", "s"),
            dimension_semantics=(pltpu.PARALLEL, pltpu.PARALLEL),
        )(x_hbm, o_hbm)

    return sck(packed, params_f, params_i, x2)


def _body(pf_ref, pi_ref, x_ref, ta_ref, tb_ref, o_ref):
    inv_s = pf_ref[0]
    two_qq = pf_ref[1]
    sh1 = pi_ref[0]
    sh2 = pi_ref[1]
    ta = ta_ref[...]
    tb = tb_ref[...]

    for i in range(0, _BLOCK_ROWS, 8):
        xv = x_ref[i:i + 8, :]
        q = jnp.clip(jnp.floor(xv * inv_s), -32768.0, 32767.0)
        qi = q.astype(jnp.int32)
        lo = (qi >> 8) & 127
        upper = qi >= 0
        r = qi & 255

        ga = jnp.take_along_axis(ta, lo, axis=1, mode="promise_in_bounds")
        gb = jnp.take_along_axis(tb, lo, axis=1, mode="promise_in_bounds")
        p = jnp.where(upper, gb, ga)

        m1 = (p << 16) >> 16
        v = (r * m1) >> sh1
        w = ((p >> 16) << sh2) + v
        o_ref[i:i + 8, :] = w.astype(jnp.float32) * two_qq


def kernel(x, c0, c1, addr_quanta, c1_quanta, addr_bits):
    del addr_bits
    orig_shape = x.shape
    n = x.size
    rows = n // _WIDTH
    x2 = x.reshape(rows, _WIDTH)

    aq = jnp.asarray(addr_quanta, jnp.int32)
    cq = jnp.asarray(c1_quanta, jnp.int32)

    def pow2i(k):
        return jax.lax.bitcast_convert_type(
            (k.astype(jnp.int32) + 127) << 23, jnp.float32)

    def mantissas(t):
        pmax = jnp.maximum(jnp.max(t), 1e-30)
        nmax = jnp.maximum(-jnp.min(t), 1e-30)
        qe = jnp.ceil(jnp.log2(jnp.maximum(pmax / 32767.5, nmax / 32768.5)))
        qe = qe.astype(jnp.int32)
        return jnp.round(t * pow2i(-qe)).astype(jnp.int32), qe

    m0, q0a = mantissas(c0)
    m1, q1a = mantissas(c1)
    packed = (m0 << 16) | (m1 & 0xFFFF)

    params_f = jnp.stack([pow2i(8 - aq), pow2i(aq + cq)])
    params_i = jnp.stack([8 + cq - q1a, q0a - aq - cq])
    tiles = [
        jnp.broadcast_to(t[None, :], (8, _LANES))
        for t in (packed[:_LANES], packed[_LANES:])
    ]

    pf16 = jnp.broadcast_to(params_f[:, None], (2, _SC_LANES))
    pi16 = jnp.broadcast_to(params_i[:, None], (2, _SC_LANES))
    return _sc_forward(x2, packed, pf16, pi16).reshape(orig_shape)

    tab_spec = pl.BlockSpec((8, _LANES), lambda i: (0, 0))
    grid = (rows // _BLOCK_ROWS,)
    y = pl.pallas_call(
        _body,
        grid=grid,
        in_specs=[
            pl.BlockSpec(memory_space=pltpu.SMEM),
            pl.BlockSpec(memory_space=pltpu.SMEM),
            pl.BlockSpec((_BLOCK_ROWS, _WIDTH), lambda i: (i, 0)),
            tab_spec,
            tab_spec,
        ],
        out_specs=pl.BlockSpec((_BLOCK_ROWS, _WIDTH), lambda i: (i, 0)),
        out_shape=jax.ShapeDtypeStruct((rows, _WIDTH), jnp.float32),
        compiler_params=pltpu.CompilerParams(
            dimension_semantics=("parallel",),
        ),
    )(params_f, params_i, x2, *tiles)
    return y.reshape(orig_shape)


# speedup vs baseline: 2.4985x; 2.4985x over previous
import jax
import jax.numpy as jnp
from jax.experimental import pallas as pl
from jax.experimental.pallas import tpu as pltpu
from jax.experimental.pallas import tpu_sc as plsc

_LANES = 128
_WIDTH = 2048
_BLOCK_ROWS = 256
_SC_LANES = 16


def _sc_forward(x2, packed, params_f, params_i, sc_rows):
    rows = sc_rows
    mesh = plsc.VectorSubcoreMesh(core_axis_name="c", subcore_axis_name="s")

    @pl.kernel(
        out_type=jax.ShapeDtypeStruct((rows, _WIDTH), jnp.float32),
        mesh=mesh,
        scratch_types=[
            pltpu.VMEM((256,), jnp.int32),
            pltpu.VMEM((2, _SC_LANES), jnp.float32),
            pltpu.VMEM((2, _SC_LANES), jnp.int32),
            pltpu.SemaphoreType.DMA,
        ],
        compiler_params=pltpu.CompilerParams(needs_layout_passes=False),
    )
    def sck(tab_hbm, pf_hbm, pi_hbm, x_hbm, o_hbm, tab_v, pf_v, pi_v, sem):
        pltpu.async_copy(tab_hbm, tab_v, sem).wait()
        pltpu.async_copy(pf_hbm, pf_v, sem).wait()
        pltpu.async_copy(pi_hbm, pi_v, sem).wait()
        inv_s = pf_v[0, :]
        two_qq = pf_v[1, :]
        sh1 = pi_v[0, :]
        sh2 = pi_v[1, :]

        def body(x_v, o_v):
            @pl.loop(0, 8)
            def _(rr):
                @pl.loop(0, _LANES, step=_SC_LANES)
                def _(cc):
                    xv = x_v[rr, pl.ds(cc, _SC_LANES)]
                    t = xv * inv_s
                    t = jnp.minimum(jnp.maximum(t, -32768.0), 32767.0)
                    qi = t.astype(jnp.int32)
                    fb = qi.astype(jnp.float32)
                    qi = jnp.where(fb > t, qi - 1, qi)
                    idx = (qi >> 8) + 128
                    r = qi & 255
                    p = plsc.load_gather(tab_v, [idx])
                    m1 = (p << 16) >> 16
                    v = (r * m1) >> sh1
                    w = ((p >> 16) << sh2) + v
                    o_v[rr, pl.ds(cc, _SC_LANES)] = w.astype(jnp.float32) * two_qq

        pltpu.emit_pipeline(
            body,
            grid=(rows // 8, _WIDTH // _LANES),
            in_specs=[pl.BlockSpec((8, _LANES), index_map=lambda i, j: (i, j))],
            out_specs=[pl.BlockSpec((8, _LANES), index_map=lambda i, j: (i, j))],
            core_axis_name=("c", "s"),
            dimension_semantics=(pltpu.PARALLEL, pltpu.PARALLEL),
        )(x_hbm, o_hbm)

    return sck(packed, params_f, params_i, x2)


_SC_ROWS = 2816


def _body(pf_ref, pi_ref, x_ref, ta_ref, tb_ref, o_ref):
    inv_s = pf_ref[0]
    two_qq = pf_ref[1]
    sh1 = pi_ref[0]
    sh2 = pi_ref[1]
    ta = ta_ref[...]
    tb = tb_ref[...]

    for i in range(0, _BLOCK_ROWS, 8):
        xv = x_ref[i:i + 8, :]
        q = jnp.clip(jnp.floor(xv * inv_s), -32768.0, 32767.0)
        qi = q.astype(jnp.int32)
        lo = (qi >> 8) & 127
        upper = qi >= 0
        r = qi & 255

        ga = jnp.take_along_axis(ta, lo, axis=1, mode="promise_in_bounds")
        gb = jnp.take_along_axis(tb, lo, axis=1, mode="promise_in_bounds")
        p = jnp.where(upper, gb, ga)

        m1 = (p << 16) >> 16
        v = (r * m1) >> sh1
        w = ((p >> 16) << sh2) + v
        o_ref[i:i + 8, :] = w.astype(jnp.float32) * two_qq


def kernel(x, c0, c1, addr_quanta, c1_quanta, addr_bits):
    del addr_bits
    orig_shape = x.shape
    n = x.size
    rows = n // _WIDTH
    x2 = x.reshape(rows, _WIDTH)

    aq = jnp.asarray(addr_quanta, jnp.int32)
    cq = jnp.asarray(c1_quanta, jnp.int32)

    def pow2i(k):
        return jax.lax.bitcast_convert_type(
            (k.astype(jnp.int32) + 127) << 23, jnp.float32)

    def mantissas(t):
        pmax = jnp.maximum(jnp.max(t), 1e-30)
        nmax = jnp.maximum(-jnp.min(t), 1e-30)
        qe = jnp.ceil(jnp.log2(jnp.maximum(pmax / 32767.5, nmax / 32768.5)))
        qe = qe.astype(jnp.int32)
        return jnp.round(t * pow2i(-qe)).astype(jnp.int32), qe

    m0, q0a = mantissas(c0)
    m1, q1a = mantissas(c1)
    packed = (m0 << 16) | (m1 & 0xFFFF)

    params_f = jnp.stack([pow2i(8 - aq), pow2i(aq + cq)])
    params_i = jnp.stack([8 + cq - q1a, q0a - aq - cq])
    tiles = [
        jnp.broadcast_to(t[None, :], (8, _LANES))
        for t in (packed[:_LANES], packed[_LANES:])
    ]

    pf16 = jnp.broadcast_to(params_f[:, None], (2, _SC_LANES))
    pi16 = jnp.broadcast_to(params_i[:, None], (2, _SC_LANES))
    y_sc = _sc_forward(x2, packed, pf16, pi16, _SC_ROWS)

    tab_spec = pl.BlockSpec((8, _LANES), lambda i: (0, 0))
    tc_rows = rows - _SC_ROWS
    off = _SC_ROWS // _BLOCK_ROWS
    grid = (tc_rows // _BLOCK_ROWS,)
    y_tc = pl.pallas_call(
        _body,
        grid=grid,
        in_specs=[
            pl.BlockSpec(memory_space=pltpu.SMEM),
            pl.BlockSpec(memory_space=pltpu.SMEM),
            pl.BlockSpec((_BLOCK_ROWS, _WIDTH), lambda i: (i + off, 0)),
            tab_spec,
            tab_spec,
        ],
        out_specs=pl.BlockSpec((_BLOCK_ROWS, _WIDTH), lambda i: (i, 0)),
        out_shape=jax.ShapeDtypeStruct((tc_rows, _WIDTH), jnp.float32),
        compiler_params=pltpu.CompilerParams(
            dimension_semantics=("parallel",),
        ),
    )(params_f, params_i, x2, *tiles)
    return jnp.concatenate([y_sc, y_tc], axis=0).reshape(orig_shape)


# speedup vs baseline: 4.7494x; 1.9009x over previous
import jax
import jax.numpy as jnp
from jax.experimental import pallas as pl
from jax.experimental.pallas import tpu as pltpu

_LANES = 128
_WIDTH = 2048
_BLOCK_ROWS = 512


def _body(pf_ref, pi_ref, x_ref, ta_ref, tb_ref, o_ref):
    inv_s = pf_ref[0]
    two_qq = pf_ref[1]
    sh1 = pi_ref[0]
    sh2 = pi_ref[1]
    ta = ta_ref[...]
    tb = tb_ref[...]

    for i in range(0, _BLOCK_ROWS, 8):
        xv = x_ref[i:i + 8, :]
        q = jnp.clip(jnp.floor(xv * inv_s), -32768.0, 32767.0)
        qi = q.astype(jnp.int32)
        lo = (qi >> 8) & 127
        upper = qi >= 0
        r = qi & 255

        ga = jnp.take_along_axis(ta, lo, axis=1, mode="promise_in_bounds")
        gb = jnp.take_along_axis(tb, lo, axis=1, mode="promise_in_bounds")
        p = jnp.where(upper, gb, ga)

        m1 = (p << 16) >> 16
        v = (r * m1) >> sh1
        w = ((p >> 16) << sh2) + v
        o_ref[i:i + 8, :] = w.astype(jnp.float32) * two_qq


def kernel(x, c0, c1, addr_quanta, c1_quanta, addr_bits):
    del addr_bits
    orig_shape = x.shape
    n = x.size
    rows = n // _WIDTH
    x2 = x.reshape(rows, _WIDTH)

    aq = jnp.asarray(addr_quanta, jnp.int32)
    cq = jnp.asarray(c1_quanta, jnp.int32)

    def pow2i(k):
        return jax.lax.bitcast_convert_type(
            (k.astype(jnp.int32) + 127) << 23, jnp.float32)

    def mantissas(t):
        pmax = jnp.maximum(jnp.max(t), 1e-30)
        nmax = jnp.maximum(-jnp.min(t), 1e-30)
        qe = jnp.ceil(jnp.log2(jnp.maximum(pmax / 32767.5, nmax / 32768.5)))
        qe = qe.astype(jnp.int32)
        return jnp.round(t * pow2i(-qe)).astype(jnp.int32), qe

    m0, q0a = mantissas(c0)
    m1, q1a = mantissas(c1)
    packed = (m0 << 16) | (m1 & 0xFFFF)

    params_f = jnp.stack([pow2i(8 - aq), pow2i(aq + cq)])
    params_i = jnp.stack([8 + cq - q1a, q0a - aq - cq])
    tiles = [
        jnp.broadcast_to(t[None, :], (8, _LANES))
        for t in (packed[:_LANES], packed[_LANES:])
    ]

    tab_spec = pl.BlockSpec((8, _LANES), lambda i: (0, 0))
    grid = (rows // _BLOCK_ROWS,)
    y = pl.pallas_call(
        _body,
        grid=grid,
        in_specs=[
            pl.BlockSpec(memory_space=pltpu.SMEM),
            pl.BlockSpec(memory_space=pltpu.SMEM),
            pl.BlockSpec((_BLOCK_ROWS, _WIDTH), lambda i: (i, 0)),
            tab_spec,
            tab_spec,
        ],
        out_specs=pl.BlockSpec((_BLOCK_ROWS, _WIDTH), lambda i: (i, 0)),
        out_shape=jax.ShapeDtypeStruct((rows, _WIDTH), jnp.float32),
        compiler_params=pltpu.CompilerParams(
            dimension_semantics=("parallel",),
        ),
    )(params_f, params_i, x2, *tiles)
    return y.reshape(orig_shape)
